# baseline (speedup 1.0000x reference)
import functools

import jax
import jax.numpy as jnp
from jax import lax
from jax.experimental import pallas as pl
from jax.experimental.pallas import tpu as pltpu
from jax.experimental.pallas import tpu_sc as plsc

_B = 512
_N = 10000
_D_IN = 256
_DIM = 512
_NUMK = 32
_BN_EPS = 1e-5
_NTOT = _B + _N
_NPAD = 10752
_CT = 512
_NC = _NPAD // _CT
_NW = 32
_RPW = _B // _NW
_NCAND = _NC * _NUMK


def _dist_body(x_ref, A_ref, be_ref, s1_ref, b1_ref, B0_ref, bl0_ref,
               B1_ref, bl1_ref, sf_ref, bf_ref,
               d_out_ref, cols_ref, u_ref, e_ref, m_ref, qs_ref):
    c = pl.program_id(0)

    h = jnp.dot(x_ref[...], A_ref[...], preferred_element_type=jnp.float32)
    h = h + be_ref[...]
    z = h * s1_ref[...] + b1_ref[...]
    z = jnp.dot(z, B0_ref[...],
                preferred_element_type=jnp.float32) + bl0_ref[...]
    z = jnp.maximum(z, 0.0)
    z = jnp.dot(z, B1_ref[...],
                preferred_element_type=jnp.float32) + bl1_ref[...]
    h = h + z
    ec = h * sf_ref[...] + bf_ref[...]
    e_ref[c] = ec

    q = e_ref[0]

    @pl.when(c == 0)
    def _qs():
        qs_ref[...] = jnp.sum(q * q, axis=1, keepdims=True)

    qs = qs_ref[...]
    cs = jnp.sum(ec * ec, axis=1)[None, :]
    p = lax.dot_general(q, ec, (((1,), (1,)), ((), ())),
                        preferred_element_type=jnp.float32)
    sq = qs + cs - 2.0 * p
    d = jnp.sqrt(jnp.maximum(sq, 1e-12))

    gcol = c * _CT + lax.broadcasted_iota(jnp.int32, (_B, _CT), 1)
    grow = lax.broadcasted_iota(jnp.int32, (_B, _CT), 0)
    bad = (gcol >= _NTOT) | (gcol == grow)
    d = jnp.where(bad, jnp.inf, d)
    d_out_ref[...] = d

    mprev = m_ref[...]
    m_ref[...] = jnp.where(c == 0, d, jnp.minimum(mprev, d))

    @pl.when(c == _NC - 1)
    def _finish():
        lane = lax.broadcasted_iota(jnp.int32, (_B, _CT), 1)
        kio = lax.broadcasted_iota(jnp.int32, (_B, _NUMK), 1)
        big = jnp.int32(2 ** 30)

        def step(k, carry):
            M, cols, _ = carry
            m = jnp.min(M, axis=1, keepdims=True)
            eq = M == m
            j = jnp.min(jnp.where(eq, lane, big), axis=1, keepdims=True)
            M = jnp.where(lane == j, jnp.inf, M)
            cols = jnp.where(kio == k, j, cols)
            return M, cols, m

        cols0 = jnp.zeros((_B, _NUMK), jnp.int32)
        u0 = jnp.zeros((_B, 1), jnp.float32)
        _, cols, u = lax.fori_loop(0, _NUMK, step, (m_ref[...], cols0, u0))
        cols_ref[...] = cols
        u_ref[...] = u


def _sc_select(d_hbm, cy_hbm, cols_hbm, u_hbm, out_hbm,
               cy_buf, row_buf, cols_buf, u_buf, out_buf, dsem, csem):
    wid = lax.axis_index("s") * 2 + lax.axis_index("c")
    base = wid * _RPW
    lane16 = lax.broadcasted_iota(jnp.int32, (16,), 0)
    lane0 = lane16 == 0
    inf16 = jnp.full((16,), jnp.inf, jnp.float32)

    pltpu.async_copy(d_hbm.at[pl.ds(base, 1)], row_buf.at[pl.ds(0, 1)], dsem)
    pltpu.async_copy(cols_hbm.at[pl.ds(base, 1)], cols_buf.at[pl.ds(0, 1)],
                     csem)
    pltpu.sync_copy(u_hbm.at[pl.ds(base, _RPW)], u_buf)
    pltpu.sync_copy(cy_hbm, cy_buf)

    def row_loop(i, _):
        row = base + i
        b = lax.rem(i, 2)
        pltpu.make_async_copy(d_hbm.at[pl.ds(row, 1)],
                              row_buf.at[pl.ds(b, 1)], dsem).wait()
        pltpu.make_async_copy(cols_hbm.at[pl.ds(row, 1)],
                              cols_buf.at[pl.ds(b, 1)], csem).wait()

        @pl.when(i + 1 < _RPW)
        def _prefetch():
            pltpu.async_copy(d_hbm.at[pl.ds(row + 1, 1)],
                             row_buf.at[pl.ds(1 - b, 1)], dsem)
            pltpu.async_copy(cols_hbm.at[pl.ds(row + 1, 1)],
                             cols_buf.at[pl.ds(1 - b, 1)], csem)

        bsplat = jnp.full((16,), b, jnp.int32)
        usplat = plsc.load_gather(u_buf, [jnp.full((16,), i, jnp.int32)])
        colv0 = plsc.load_gather(cols_buf, [bsplat, lane16])
        colv1 = plsc.load_gather(cols_buf, [bsplat, lane16 + 16])

        def stream(p, carry):
            for colv in (colv0, colv1):
                s0k, s0v, s1k, s1v = carry
                idx = colv + p * _CT
                dv = plsc.load_gather(row_buf, [bsplat, idx])
                keep = dv <= usplat
                cnt = plsc.all_reduce_population_count(keep)[0]

                def merge(op):
                    s0k, s0v, s1k, s1v, dv, idx = op
                    bk, bv = plsc.sort_key_val(dv, idx)
                    rbk = lax.rev(bk, (0,))
                    rbv = lax.rev(bv, (0,))
                    cm = s1k <= rbk
                    l1k = jnp.where(cm, s1k, rbk)
                    l1v = jnp.where(cm, s1v, rbv)
                    c2 = s0k <= l1k
                    m0k = jnp.where(c2, s0k, l1k)
                    m0v = jnp.where(c2, s0v, l1v)
                    m1k = jnp.where(c2, l1k, s0k)
                    m1v = jnp.where(c2, l1v, s0v)
                    n0k, n0v = plsc.sort_key_val(m0k, m0v)
                    n1k, n1v = plsc.sort_key_val(m1k, m1v)
                    return n0k, n0v, n1k, n1v

                def skip(op):
                    s0k, s0v, s1k, s1v, _, _ = op
                    return s0k, s0v, s1k, s1v

                carry = lax.cond(cnt > 0, merge, skip,
                                 (s0k, s0v, s1k, s1v, dv, idx))
            return carry

        zi = jnp.zeros((16,), jnp.int32)
        s0k, s0v, s1k, s1v = lax.fori_loop(
            0, _NC, stream, (inf16, zi, inf16, zi))

        d1 = jnp.min(s0k)
        d1s = jnp.broadcast_to(d1, (16,))
        w0 = jnp.exp(d1s - s0k)
        w1 = jnp.exp(d1s - s1k)
        cy0 = plsc.load_gather(cy_buf, [s0v])
        cy1 = plsc.load_gather(cy_buf, [s1v])
        num = jnp.broadcast_to(jnp.sum(w0 * cy0 + w1 * cy1), (16,))
        den = jnp.broadcast_to(jnp.sum(w0 + w1), (16,))
        res = num / den
        plsc.store_scatter(out_buf, [jnp.full((16,), i, jnp.int32)], res,
                           mask=lane0)
        return 0

    lax.fori_loop(0, _RPW, row_loop, 0)
    pltpu.sync_copy(out_buf, out_hbm.at[pl.ds(base, _RPW)])


def kernel(x, y, candidate_x, candidate_y, context_size, is_train,
           enc_W, enc_b, bn1_g, bn1_b, l0_W, l0_b, l1_W, l1_b, bnf_g, bnf_b):
    inv = 1.0 / jnp.sqrt(1.0 + _BN_EPS)
    s1 = (bn1_g * inv)[None, :]
    sf = (bnf_g * inv)[None, :]

    xall = jnp.concatenate([x, candidate_x], axis=0)
    xall = jnp.pad(xall, ((0, _NPAD - _NTOT), (0, 0)))

    d, cols, u = pl.pallas_call(
        _dist_body,
        grid=(_NC,),
        in_specs=[
            pl.BlockSpec((_CT, _D_IN), lambda c: (c, 0)),
            pl.BlockSpec((_D_IN, _DIM), lambda c: (0, 0)),
            pl.BlockSpec((1, _DIM), lambda c: (0, 0)),
            pl.BlockSpec((1, _DIM), lambda c: (0, 0)),
            pl.BlockSpec((1, _DIM), lambda c: (0, 0)),
            pl.BlockSpec((_DIM, _DIM), lambda c: (0, 0)),
            pl.BlockSpec((1, _DIM), lambda c: (0, 0)),
            pl.BlockSpec((_DIM, _DIM), lambda c: (0, 0)),
            pl.BlockSpec((1, _DIM), lambda c: (0, 0)),
            pl.BlockSpec((1, _DIM), lambda c: (0, 0)),
            pl.BlockSpec((1, _DIM), lambda c: (0, 0)),
        ],
        out_specs=[
            pl.BlockSpec((_B, _CT), lambda c: (0, c)),
            pl.BlockSpec((_B, _NUMK), lambda c: (0, 0)),
            pl.BlockSpec((_B, 1), lambda c: (0, 0)),
        ],
        out_shape=[
            jax.ShapeDtypeStruct((_B, _NPAD), jnp.float32),
            jax.ShapeDtypeStruct((_B, _NUMK), jnp.int32),
            jax.ShapeDtypeStruct((_B, 1), jnp.float32),
        ],
        scratch_shapes=[
            pltpu.VMEM((_NC, _CT, _DIM), jnp.float32),
            pltpu.VMEM((_B, _CT), jnp.float32),
            pltpu.VMEM((_B, 1), jnp.float32),
        ],
        compiler_params=pltpu.CompilerParams(
            dimension_semantics=("arbitrary",)),
    )(xall, enc_W.T, enc_b[None, :], s1, bn1_b[None, :], l0_W.T,
      l0_b[None, :], l1_W.T, l1_b[None, :], sf, bnf_b[None, :])

    cy = jnp.concatenate([y, candidate_y], axis=0)
    cy = jnp.pad(cy, (0, _NPAD - _NTOT))

    sc = pl.kernel(
        _sc_select,
        out_type=jax.ShapeDtypeStruct((_B,), jnp.float32),
        mesh=plsc.VectorSubcoreMesh(core_axis_name="c", subcore_axis_name="s"),
        scratch_types=[
            pltpu.VMEM((_NPAD,), jnp.float32),
            pltpu.VMEM((2, _NPAD), jnp.float32),
            pltpu.VMEM((2, _NUMK), jnp.int32),
            pltpu.VMEM((_RPW,), jnp.float32),
            pltpu.VMEM((_RPW,), jnp.float32),
            pltpu.SemaphoreType.DMA,
            pltpu.SemaphoreType.DMA,
        ],
        compiler_params=pltpu.CompilerParams(needs_layout_passes=False),
    )
    return sc(d, cy, cols, u[:, 0])


# speedup vs baseline: 1.0201x; 1.0201x over previous
import functools

import jax
import jax.numpy as jnp
from jax import lax
from jax.experimental import pallas as pl
from jax.experimental.pallas import tpu as pltpu
from jax.experimental.pallas import tpu_sc as plsc

_B = 512
_N = 10000
_D_IN = 256
_DIM = 512
_NUMK = 32
_BN_EPS = 1e-5
_NTOT = _B + _N
_NPAD = 10752
_CT = 512
_NC = _NPAD // _CT
_NW = 32
_RPW = _B // _NW
_NCAND = _NC * _NUMK


def _dist_body(x_ref, A_ref, be_ref, s1_ref, b1_ref, B0_ref, bl0_ref,
               B1_ref, bl1_ref, sf_ref, bf_ref,
               d_out_ref, cols_ref, u_ref, e_ref, m_ref):
    c = pl.program_id(0)

    h = jnp.dot(x_ref[...], A_ref[...], preferred_element_type=jnp.float32)
    h = h + be_ref[...]
    z = h * s1_ref[...] + b1_ref[...]
    z = jnp.dot(z, B0_ref[...],
                preferred_element_type=jnp.float32) + bl0_ref[...]
    z = jnp.maximum(z, 0.0)
    z = jnp.dot(z, B1_ref[...],
                preferred_element_type=jnp.float32) + bl1_ref[...]
    h = h + z
    ec = h * sf_ref[...] + bf_ref[...]
    e_ref[c] = ec

    q = e_ref[0]
    qs = jnp.sum(q * q, axis=1, keepdims=True)
    cs = jnp.sum(ec * ec, axis=1)[None, :]
    p = lax.dot_general(q, ec, (((1,), (1,)), ((), ())),
                        preferred_element_type=jnp.float32)
    sq = qs + cs - 2.0 * p
    d = jnp.sqrt(jnp.maximum(sq, 1e-12))

    gcol = c * _CT + lax.broadcasted_iota(jnp.int32, (_B, _CT), 1)
    grow = lax.broadcasted_iota(jnp.int32, (_B, _CT), 0)
    bad = (gcol >= _NTOT) | (gcol == grow)
    d = jnp.where(bad, jnp.inf, d)
    d_out_ref[...] = d

    mprev = m_ref[...]
    m_ref[...] = jnp.where(c == 0, d, jnp.minimum(mprev, d))

    @pl.when(c == _NC - 1)
    def _finish():
        lane = lax.broadcasted_iota(jnp.int32, (_B, _CT), 1)
        kio = lax.broadcasted_iota(jnp.int32, (_B, _NUMK), 1)
        big = jnp.int32(2 ** 30)

        def step(k, carry):
            M, cols, _ = carry
            m = jnp.min(M, axis=1, keepdims=True)
            eq = M == m
            j = jnp.min(jnp.where(eq, lane, big), axis=1, keepdims=True)
            M = jnp.where(lane == j, jnp.inf, M)
            cols = jnp.where(kio == k, j, cols)
            return M, cols, m

        cols0 = jnp.zeros((_B, _NUMK), jnp.int32)
        u0 = jnp.zeros((_B, 1), jnp.float32)
        _, cols, u = lax.fori_loop(0, _NUMK, step, (m_ref[...], cols0, u0))
        cols_ref[...] = cols
        u_ref[...] = u


def _sc_select(d_hbm, cy_hbm, cols_hbm, u_hbm, out_hbm,
               cy_buf, row_buf, cols_buf, u_buf, out_buf, dsem, csem):
    wid = lax.axis_index("s") * 2 + lax.axis_index("c")
    base = wid * _RPW
    lane16 = lax.broadcasted_iota(jnp.int32, (16,), 0)
    lane0 = lane16 == 0
    inf16 = jnp.full((16,), jnp.inf, jnp.float32)

    pltpu.async_copy(d_hbm.at[pl.ds(base, 1)], row_buf.at[pl.ds(0, 1)], dsem)
    pltpu.async_copy(cols_hbm.at[pl.ds(base, 1)], cols_buf.at[pl.ds(0, 1)],
                     csem)
    pltpu.sync_copy(u_hbm.at[pl.ds(base, _RPW)], u_buf)
    pltpu.sync_copy(cy_hbm, cy_buf)

    def row_loop(i, _):
        row = base + i
        b = lax.rem(i, 2)
        pltpu.make_async_copy(d_hbm.at[pl.ds(row, 1)],
                              row_buf.at[pl.ds(b, 1)], dsem).wait()
        pltpu.make_async_copy(cols_hbm.at[pl.ds(row, 1)],
                              cols_buf.at[pl.ds(b, 1)], csem).wait()

        @pl.when(i + 1 < _RPW)
        def _prefetch():
            pltpu.async_copy(d_hbm.at[pl.ds(row + 1, 1)],
                             row_buf.at[pl.ds(1 - b, 1)], dsem)
            pltpu.async_copy(cols_hbm.at[pl.ds(row + 1, 1)],
                             cols_buf.at[pl.ds(1 - b, 1)], csem)

        bsplat = jnp.full((16,), b, jnp.int32)
        usplat = plsc.load_gather(u_buf, [jnp.full((16,), i, jnp.int32)])

        def stream(p, carry):
            for h in range(2):
                s0k, s0v, s1k, s1v = carry
                colv = plsc.load_gather(cols_buf, [bsplat, lane16 + h * 16])
                idx = colv + p * _CT
                dv = plsc.load_gather(row_buf, [bsplat, idx])
                keep = dv <= usplat
                cnt = plsc.all_reduce_population_count(keep)[0]

                def merge(op):
                    s0k, s0v, s1k, s1v, dv, idx = op
                    bk, bv = plsc.sort_key_val(dv, idx)
                    rbk = lax.rev(bk, (0,))
                    rbv = lax.rev(bv, (0,))
                    cm = s1k <= rbk
                    l1k = jnp.where(cm, s1k, rbk)
                    l1v = jnp.where(cm, s1v, rbv)
                    c2 = s0k <= l1k
                    m0k = jnp.where(c2, s0k, l1k)
                    m0v = jnp.where(c2, s0v, l1v)
                    m1k = jnp.where(c2, l1k, s0k)
                    m1v = jnp.where(c2, l1v, s0v)
                    n0k, n0v = plsc.sort_key_val(m0k, m0v)
                    n1k, n1v = plsc.sort_key_val(m1k, m1v)
                    return n0k, n0v, n1k, n1v

                def skip(op):
                    s0k, s0v, s1k, s1v, _, _ = op
                    return s0k, s0v, s1k, s1v

                carry = lax.cond(cnt > 0, merge, skip,
                                 (s0k, s0v, s1k, s1v, dv, idx))
            return carry

        zi = jnp.zeros((16,), jnp.int32)
        s0k, s0v, s1k, s1v = lax.fori_loop(
            0, _NC, stream, (inf16, zi, inf16, zi))

        d1 = jnp.min(s0k)
        d1s = jnp.broadcast_to(d1, (16,))
        w0 = jnp.exp(d1s - s0k)
        w1 = jnp.exp(d1s - s1k)
        cy0 = plsc.load_gather(cy_buf, [s0v])
        cy1 = plsc.load_gather(cy_buf, [s1v])
        num = jnp.broadcast_to(jnp.sum(w0 * cy0 + w1 * cy1), (16,))
        den = jnp.broadcast_to(jnp.sum(w0 + w1), (16,))
        res = num / den
        plsc.store_scatter(out_buf, [jnp.full((16,), i, jnp.int32)], res,
                           mask=lane0)
        return 0

    lax.fori_loop(0, _RPW, row_loop, 0)
    pltpu.sync_copy(out_buf, out_hbm.at[pl.ds(base, _RPW)])


def kernel(x, y, candidate_x, candidate_y, context_size, is_train,
           enc_W, enc_b, bn1_g, bn1_b, l0_W, l0_b, l1_W, l1_b, bnf_g, bnf_b):
    inv = 1.0 / jnp.sqrt(1.0 + _BN_EPS)
    s1 = (bn1_g * inv)[None, :]
    sf = (bnf_g * inv)[None, :]

    xall = jnp.concatenate([x, candidate_x], axis=0)
    xall = jnp.pad(xall, ((0, _NPAD - _NTOT), (0, 0)))

    d, cols, u = pl.pallas_call(
        _dist_body,
        grid=(_NC,),
        in_specs=[
            pl.BlockSpec((_CT, _D_IN), lambda c: (c, 0)),
            pl.BlockSpec((_D_IN, _DIM), lambda c: (0, 0)),
            pl.BlockSpec((1, _DIM), lambda c: (0, 0)),
            pl.BlockSpec((1, _DIM), lambda c: (0, 0)),
            pl.BlockSpec((1, _DIM), lambda c: (0, 0)),
            pl.BlockSpec((_DIM, _DIM), lambda c: (0, 0)),
            pl.BlockSpec((1, _DIM), lambda c: (0, 0)),
            pl.BlockSpec((_DIM, _DIM), lambda c: (0, 0)),
            pl.BlockSpec((1, _DIM), lambda c: (0, 0)),
            pl.BlockSpec((1, _DIM), lambda c: (0, 0)),
            pl.BlockSpec((1, _DIM), lambda c: (0, 0)),
        ],
        out_specs=[
            pl.BlockSpec((_B, _CT), lambda c: (0, c)),
            pl.BlockSpec((_B, _NUMK), lambda c: (0, 0)),
            pl.BlockSpec((_B, 1), lambda c: (0, 0)),
        ],
        out_shape=[
            jax.ShapeDtypeStruct((_B, _NPAD), jnp.float32),
            jax.ShapeDtypeStruct((_B, _NUMK), jnp.int32),
            jax.ShapeDtypeStruct((_B, 1), jnp.float32),
        ],
        scratch_shapes=[
            pltpu.VMEM((_NC, _CT, _DIM), jnp.float32),
            pltpu.VMEM((_B, _CT), jnp.float32),
        ],
        compiler_params=pltpu.CompilerParams(
            dimension_semantics=("arbitrary",)),
    )(xall, enc_W.T, enc_b[None, :], s1, bn1_b[None, :], l0_W.T,
      l0_b[None, :], l1_W.T, l1_b[None, :], sf, bnf_b[None, :])

    cy = jnp.concatenate([y, candidate_y], axis=0)
    cy = jnp.pad(cy, (0, _NPAD - _NTOT))

    sc = pl.kernel(
        _sc_select,
        out_type=jax.ShapeDtypeStruct((_B,), jnp.float32),
        mesh=plsc.VectorSubcoreMesh(core_axis_name="c", subcore_axis_name="s"),
        scratch_types=[
            pltpu.VMEM((_NPAD,), jnp.float32),
            pltpu.VMEM((2, _NPAD), jnp.float32),
            pltpu.VMEM((2, _NUMK), jnp.int32),
            pltpu.VMEM((_RPW,), jnp.float32),
            pltpu.VMEM((_RPW,), jnp.float32),
            pltpu.SemaphoreType.DMA,
            pltpu.SemaphoreType.DMA,
        ],
        compiler_params=pltpu.CompilerParams(needs_layout_passes=False),
    )
    return sc(d, cy, cols, u[:, 0])
